# Initial kernel scaffold; baseline (speedup 1.0000x reference)
import jax
import jax.numpy as jnp
from jax.experimental import pallas as pl


def kernel(x, edge_index, batch, params):
    raise NotImplementedError("write your pallas kernel here")


# speedup vs baseline: 7.4006x; 7.4006x over previous
import functools

import jax
import jax.numpy as jnp
from jax import lax
from jax.experimental import pallas as pl
from jax.experimental.pallas import tpu as pltpu
from jax.experimental.pallas import tpu_sc as plsc

_N = 10000
_E = 320000
_H = 128
_G = 64
_NC, _NS = 2, 16
_NW = _NC * _NS
_CH = 125
_CPW = _E // _NW // _CH
_RPT = 624
_ZCH = 104
_TAIL = _N - _NS * _RPT


def _sc_agg_body(h_hbm, src_hbm, dst_hbm, out_hbm, src_v, dst_v, rows_v,
                 acc, sem):
    cid = lax.axis_index("c")
    sid = lax.axis_index("s")
    wid = cid * _NS + sid

    def _zb(r, carry):
        for cc in range(_H // 16):
            rows_v[r, pl.ds(cc * 16, 16)] = jnp.zeros((16,), jnp.float32)
        return carry

    lax.fori_loop(0, _ZCH, _zb, 0)
    for k in range(_RPT // _ZCH):
        pltpu.sync_copy(rows_v.at[pl.ds(0, _ZCH)],
                        acc.at[pl.ds(sid * _RPT + k * _ZCH, _ZCH)])

    @pl.when(sid == _NS - 1)
    def _zero_tail():
        pltpu.sync_copy(rows_v.at[pl.ds(0, _TAIL)],
                        acc.at[pl.ds(_NS * _RPT, _TAIL)])

    pltpu.sync_copy(src_hbm.at[wid], src_v)
    pltpu.sync_copy(dst_hbm.at[wid], dst_v)
    plsc.subcore_barrier()

    def _eb(j, carry):
        pltpu.async_copy(h_hbm.at[src_v.at[j]], rows_v, sem).wait()
        pltpu.sync_copy(rows_v, acc.at[dst_v.at[j]], add=True)
        return carry

    lax.fori_loop(0, _CPW, _eb, 0)
    plsc.subcore_barrier()
    pltpu.sync_copy(acc.at[pl.ds(sid * _RPT, _RPT)],
                    out_hbm.at[cid, pl.ds(sid * _RPT, _RPT)])

    @pl.when(sid == _NS - 1)
    def _write_tail():
        pltpu.sync_copy(acc.at[pl.ds(_NS * _RPT, _TAIL)],
                        out_hbm.at[cid, pl.ds(_NS * _RPT, _TAIL)])


@functools.cache
def _get_sc_agg():
    return pl.kernel(
        _sc_agg_body,
        out_type=jax.ShapeDtypeStruct((_NC, _N, _H), jnp.float32),
        mesh=plsc.VectorSubcoreMesh(core_axis_name="c", subcore_axis_name="s"),
        scratch_types=[
            pltpu.VMEM((_CPW, _CH), jnp.int32),
            pltpu.VMEM((_CPW, _CH), jnp.int32),
            pltpu.VMEM((_CH, _H), jnp.float32),
            pltpu.VMEM_SHARED((_N, _H), jnp.float32),
            pltpu.SemaphoreType.DMA,
        ],
    )


def _bdot(a, b):
    return jnp.dot(a.astype(jnp.bfloat16), b.astype(jnp.bfloat16),
                   preferred_element_type=jnp.float32)


_HALF_VREGS = _N // 16
_VPB = 25


def _sub_tree(acc):
    t = acc[0:4] + acc[4:8]
    t = t[0:2] + t[2:4]
    return t[0:1] + t[1:2]


def _half_reduce(scr_ref, base, elem):
    def body(i, acc):
        blk = scr_ref[pl.ds(base + i * (_VPB * 8), _VPB * 8), :]
        for k in range(_VPB):
            acc = acc + elem(blk[k * 8:(k + 1) * 8])
        return acc

    acc = lax.fori_loop(0, _HALF_VREGS // _VPB, body,
                        jnp.zeros((8, _H), jnp.float32))
    return _sub_tree(acc)


def _colmean(scr_ref):
    def body(i, acc):
        blk = scr_ref[pl.ds(i * (_VPB * 8), _VPB * 8), :]
        for k in range(_VPB):
            acc = acc + blk[k * 8:(k + 1) * 8]
        return acc

    acc = lax.fori_loop(0, _N // (_VPB * 8), body,
                        jnp.zeros((8, _H), jnp.float32))
    return _sub_tree(acc) * (1.0 / _N)


def _colvar(scr_ref, m):
    def elem(v):
        d = v - m
        return d * d
    s = _half_reduce(scr_ref, 0, elem) + _half_reduce(scr_ref, _N // 2, elem)
    return s * (1.0 / _N)


def _bn_block(scr_ref, z, g, b):
    scr_ref[...] = z
    m = _colmean(scr_ref)
    v = _colvar(scr_ref, m)
    z = (scr_ref[...] - m) / jnp.sqrt(v + 1e-5) * g + b
    return jnp.maximum(z, 0.0)


def _dense_z(scr_ref, h, agg0, agg1, eps, w1, b1, g1, bb1, w2, b2, g2, bb2):
    z = (1.0 + eps) * h + (agg0 + agg1)
    z = _bn_block(scr_ref, _bdot(z, w1) + b1, g1, bb1)
    z = _bn_block(scr_ref, _bdot(z, w2) + b2, g2, bb2)
    return z


def _pool(bat, z):
    onehot = (bat == lax.broadcasted_iota(jnp.int32, (_N, _G), 1)).astype(jnp.float32)
    return lax.dot_general(onehot, z, (((0,), (0,)), ((), ())),
                           preferred_element_type=jnp.float32,
                           precision=lax.Precision.HIGHEST)


def _tc_layer_body(h_ref, agg_ref, eps_ref, w1_ref, b1_ref, g1_ref, bb1_ref,
                   w2_ref, b2_ref, g2_ref, bb2_ref, bat_ref, z_ref, pool_ref,
                   scr_ref):
    z = _dense_z(scr_ref, h_ref[...], agg_ref[0], agg_ref[1], eps_ref[0, 0],
                 w1_ref[...], b1_ref[...], g1_ref[...], bb1_ref[...],
                 w2_ref[...], b2_ref[...], g2_ref[...], bb2_ref[...])
    z_ref[...] = z
    pool_ref[...] = _pool(bat_ref[...], z)


_tc_layer = pl.pallas_call(
    _tc_layer_body,
    out_shape=(jax.ShapeDtypeStruct((_N, _H), jnp.float32),
               jax.ShapeDtypeStruct((_G, _H), jnp.float32)),
    scratch_shapes=[pltpu.VMEM((_N, _H), jnp.float32)],
)


def _tc_final_body(h_ref, agg_ref, eps_ref, w1_ref, b1_ref, g1_ref, bb1_ref,
                   w2_ref, b2_ref, g2_ref, bb2_ref, bat_ref, p1_ref, p2_ref,
                   cw1_ref, cb1_ref, cw2_ref, cb2_ref, cw3_ref, cb3_ref,
                   out_ref, scr_ref):
    z = _dense_z(scr_ref, h_ref[...], agg_ref[0], agg_ref[1], eps_ref[0, 0],
                 w1_ref[...], b1_ref[...], g1_ref[...], bb1_ref[...],
                 w2_ref[...], b2_ref[...], g2_ref[...], bb2_ref[...])
    p3 = _pool(bat_ref[...], z)
    g = jnp.concatenate([p1_ref[...], p2_ref[...], p3], axis=-1)
    g = _bdot(g, cw1_ref[...]) + cb1_ref[...]
    g = jnp.maximum(g, 0.0)
    g = _bdot(g, cw2_ref[...]) + cb2_ref[...]
    g = jnp.maximum(g, 0.0)
    g = _bdot(g, cw3_ref[...]) + cb3_ref[...]
    out_ref[...] = g


_tc_final = pl.pallas_call(
    _tc_final_body,
    out_shape=jax.ShapeDtypeStruct((_G, 1), jnp.float32),
    scratch_shapes=[pltpu.VMEM((_N, _H), jnp.float32)],
)


def kernel(x, edge_index, batch, params):
    src3 = edge_index[0].reshape(_NW, _CPW, _CH)
    dst3 = edge_index[1].reshape(_NW, _CPW, _CH)
    bat = batch.reshape(_N, 1)

    def row(a):
        return a.reshape(1, -1)

    h = x
    pooled = []
    for i in range(3):
        c = params["convs"][i]
        bn = params["bns"][i]
        agg = _get_sc_agg()(h, src3, dst3)
        args = (h, agg, c["eps"].reshape(1, 1), c["W1"], row(c["b1"]),
                row(c["bn1_g"]), row(c["bn1_b"]), c["W2"], row(c["b2"]),
                row(bn["g"]), row(bn["b"]), bat)
        if i < 2:
            h, p = _tc_layer(*args)
            pooled.append(p)
        else:
            cl = params["cls"]
            out = _tc_final(*args, pooled[0], pooled[1],
                            cl["W1"], row(cl["b1"]), cl["W2"], row(cl["b2"]),
                            cl["W3"], cl["b3"].reshape(1, 1))
    return jnp.squeeze(out, -1)
